# baseline (speedup 1.0000x reference)
import functools

import jax
import jax.numpy as jnp
from jax.experimental import pallas as pl
from jax.experimental.pallas import tpu as pltpu


def _largest_divisor(n, cap):
    for d in range(min(n, cap), 0, -1):
        if n % d == 0 and d % 8 == 0:
            return d
    return n


def _support_kernel(x_ref, w_ref, out_ref):
    out_ref[...] = jnp.dot(x_ref[...], w_ref[...],
                           preferred_element_type=jnp.float32)


def _gcn_kernel(fadj_ref, sup_ref, bgc_ref, wfc_ref, bfc_ref, out_ref):
    h = jnp.dot(fadj_ref[...], sup_ref[...],
                preferred_element_type=jnp.float32) + bgc_ref[...]
    h = jnp.where(h > 0, h, jnp.exp(jnp.minimum(h, 0.0)) - 1.0)
    out_ref[...] = (
        jnp.dot(h, wfc_ref[...], preferred_element_type=jnp.float32)
        + bfc_ref[...]
    )


@jax.jit
def kernel(input, fadj, W_gc, b_gc, W_fc, b_fc):
    n, n_in = input.shape
    nfea = W_gc.shape[1]
    n_class = W_fc.shape[1]

    bm_s = _largest_divisor(n, 1000)
    support = pl.pallas_call(
        _support_kernel,
        grid=(n // bm_s,),
        in_specs=[
            pl.BlockSpec((bm_s, n_in), lambda i: (i, 0)),
            pl.BlockSpec((n_in, nfea), lambda i: (0, 0)),
        ],
        out_specs=pl.BlockSpec((bm_s, nfea), lambda i: (i, 0)),
        out_shape=jax.ShapeDtypeStruct((n, nfea), jnp.float32),
    )(input, W_gc)

    bm = _largest_divisor(n, 400)

    out = pl.pallas_call(
        _gcn_kernel,
        grid=(n // bm,),
        in_specs=[
            pl.BlockSpec((bm, n), lambda i: (i, 0)),
            pl.BlockSpec((n, nfea), lambda i: (0, 0)),
            pl.BlockSpec((1, nfea), lambda i: (0, 0)),
            pl.BlockSpec((nfea, n_class), lambda i: (0, 0)),
            pl.BlockSpec((1, n_class), lambda i: (0, 0)),
        ],
        out_specs=pl.BlockSpec((bm, n_class), lambda i: (i, 0)),
        out_shape=jax.ShapeDtypeStruct((n, n_class), jnp.float32),
        compiler_params=pltpu.CompilerParams(
            dimension_semantics=("arbitrary",),
        ),
    )(fadj, support, b_gc.reshape(1, nfea), W_fc, b_fc.reshape(1, n_class))

    return out


# speedup vs baseline: 1.0006x; 1.0006x over previous
import functools

import jax
import jax.numpy as jnp
from jax.experimental import pallas as pl
from jax.experimental.pallas import tpu as pltpu


def _largest_divisor(n, cap):
    for d in range(min(n, cap), 0, -1):
        if n % d == 0 and d % 8 == 0:
            return d
    return n


def _support_kernel(x_ref, w_ref, out_ref):
    out_ref[...] = jnp.dot(x_ref[...], w_ref[...],
                           preferred_element_type=jnp.float32)


def _gcn_kernel(fadj_ref, sup_ref, bgc_ref, wfc_ref, bfc_ref, out_ref):
    h = jnp.dot(fadj_ref[...], sup_ref[...],
                preferred_element_type=jnp.float32) + bgc_ref[...]
    h = jnp.where(h > 0, h, jnp.exp(jnp.minimum(h, 0.0)) - 1.0)
    out_ref[...] = (
        jnp.dot(h, wfc_ref[...], preferred_element_type=jnp.float32)
        + bfc_ref[...]
    )


@jax.jit
def kernel(input, fadj, W_gc, b_gc, W_fc, b_fc):
    n, n_in = input.shape
    nfea = W_gc.shape[1]
    n_class = W_fc.shape[1]

    bm_s = _largest_divisor(n, 1000)
    support = pl.pallas_call(
        _support_kernel,
        grid=(n // bm_s,),
        in_specs=[
            pl.BlockSpec((bm_s, n_in), lambda i: (i, 0)),
            pl.BlockSpec((n_in, nfea), lambda i: (0, 0)),
        ],
        out_specs=pl.BlockSpec((bm_s, nfea), lambda i: (i, 0)),
        out_shape=jax.ShapeDtypeStruct((n, nfea), jnp.float32),
    )(input, W_gc)

    bm = _largest_divisor(n, 400)

    out = pl.pallas_call(
        _gcn_kernel,
        grid=(n // bm,),
        in_specs=[
            pl.BlockSpec((bm, n), lambda i: (i, 0)),
            pl.BlockSpec((n, nfea), lambda i: (0, 0)),
            pl.BlockSpec((1, nfea), lambda i: (0, 0)),
            pl.BlockSpec((nfea, n_class), lambda i: (0, 0)),
            pl.BlockSpec((1, n_class), lambda i: (0, 0)),
        ],
        out_specs=pl.BlockSpec((bm, n_class), lambda i: (i, 0)),
        out_shape=jax.ShapeDtypeStruct((n, n_class), jnp.float32),
        compiler_params=pltpu.CompilerParams(
            dimension_semantics=("parallel",),
        ),
    )(fadj, support, b_gc.reshape(1, nfea), W_fc, b_fc.reshape(1, n_class))

    return out


# speedup vs baseline: 1.0859x; 1.0853x over previous
import jax
import jax.numpy as jnp
from jax.experimental import pallas as pl
from jax.experimental.pallas import tpu as pltpu


def _largest_divisor(n, cap):
    for d in range(min(n, cap), 0, -1):
        if n % d == 0 and d % 8 == 0:
            return d
    return n


def _gcn_kernel(x_ref, wgc_ref, fadj_ref, bgc_ref, wfc_ref, bfc_ref,
                out_ref, sup_ref):
    @pl.when(pl.program_id(0) == 0)
    def _():
        sup_ref[...] = jnp.dot(x_ref[...], wgc_ref[...],
                               preferred_element_type=jnp.float32)

    h = jnp.dot(fadj_ref[...], sup_ref[...],
                preferred_element_type=jnp.float32) + bgc_ref[...]
    h = jnp.where(h > 0, h, jnp.exp(jnp.minimum(h, 0.0)) - 1.0)
    out_ref[...] = (
        jnp.dot(h, wfc_ref[...], preferred_element_type=jnp.float32)
        + bfc_ref[...]
    )


@jax.jit
def kernel(input, fadj, W_gc, b_gc, W_fc, b_fc):
    n, n_in = input.shape
    nfea = W_gc.shape[1]
    n_class = W_fc.shape[1]

    bm = _largest_divisor(n, 400)

    out = pl.pallas_call(
        _gcn_kernel,
        grid=(n // bm,),
        in_specs=[
            pl.BlockSpec((n, n_in), lambda i: (0, 0)),
            pl.BlockSpec((n_in, nfea), lambda i: (0, 0)),
            pl.BlockSpec((bm, n), lambda i: (i, 0)),
            pl.BlockSpec((1, nfea), lambda i: (0, 0)),
            pl.BlockSpec((nfea, n_class), lambda i: (0, 0)),
            pl.BlockSpec((1, n_class), lambda i: (0, 0)),
        ],
        out_specs=pl.BlockSpec((bm, n_class), lambda i: (i, 0)),
        out_shape=jax.ShapeDtypeStruct((n, n_class), jnp.float32),
        scratch_shapes=[pltpu.VMEM((n, nfea), jnp.float32)],
        compiler_params=pltpu.CompilerParams(
            dimension_semantics=("arbitrary",),
        ),
    )(input, W_gc, fadj, b_gc.reshape(1, nfea), W_fc,
      b_fc.reshape(1, n_class))

    return out


# speedup vs baseline: 1.0923x; 1.0059x over previous
import jax
import jax.numpy as jnp
from jax.experimental import pallas as pl
from jax.experimental.pallas import tpu as pltpu


def _largest_divisor(n, cap):
    for d in range(min(n, cap), 0, -1):
        if n % d == 0 and d % 8 == 0:
            return d
    return n


def _gcn_kernel(x_ref, wgc_ref, fadj_ref, bgc_ref, wfc_ref, bfc_ref,
                out_ref, sup_ref):
    @pl.when(pl.program_id(0) == 0)
    def _():
        sup_ref[...] = jnp.dot(x_ref[...], wgc_ref[...],
                               preferred_element_type=jnp.float32
                               ).astype(jnp.bfloat16)

    h = jnp.dot(fadj_ref[...].astype(jnp.bfloat16), sup_ref[...],
                preferred_element_type=jnp.float32) + bgc_ref[...]
    h = jnp.where(h > 0, h, jnp.exp(jnp.minimum(h, 0.0)) - 1.0)
    out_ref[...] = (
        jnp.dot(h, wfc_ref[...], preferred_element_type=jnp.float32)
        + bfc_ref[...]
    )


@jax.jit
def kernel(input, fadj, W_gc, b_gc, W_fc, b_fc):
    n, n_in = input.shape
    nfea = W_gc.shape[1]
    n_class = W_fc.shape[1]

    bm = _largest_divisor(n, 400)

    out = pl.pallas_call(
        _gcn_kernel,
        grid=(n // bm,),
        in_specs=[
            pl.BlockSpec((n, n_in), lambda i: (0, 0)),
            pl.BlockSpec((n_in, nfea), lambda i: (0, 0)),
            pl.BlockSpec((bm, n), lambda i: (i, 0)),
            pl.BlockSpec((1, nfea), lambda i: (0, 0)),
            pl.BlockSpec((nfea, n_class), lambda i: (0, 0)),
            pl.BlockSpec((1, n_class), lambda i: (0, 0)),
        ],
        out_specs=pl.BlockSpec((bm, n_class), lambda i: (i, 0)),
        out_shape=jax.ShapeDtypeStruct((n, n_class), jnp.float32),
        scratch_shapes=[pltpu.VMEM((n, nfea), jnp.bfloat16)],
        compiler_params=pltpu.CompilerParams(
            dimension_semantics=("arbitrary",),
        ),
    )(input, W_gc, fadj, b_gc.reshape(1, nfea), W_fc,
      b_fc.reshape(1, n_class))

    return out
